# Initial kernel scaffold; baseline (speedup 1.0000x reference)
import jax
import jax.numpy as jnp
from jax.experimental import pallas as pl


def kernel(u_t, memory_state, read_ema, W_q, W_k, W_v):
    raise NotImplementedError("write your pallas kernel here")


# speedup vs baseline: 3.5292x; 3.5292x over previous
import functools

import jax
import jax.numpy as jnp
from jax.experimental import pallas as pl
from jax.experimental.pallas import tpu as pltpu

B = 1024
J = 2048
INPUT_DIM = 1792
D_M = 64
H = 8
HD = 16
D_Q = H * HD
D_V = 64
VHD = D_V // H
KAPPA = 8
LOAD_PENALTY = 0.01

BT = 16
R = BT * H


def _router_kernel(u_ref, ema_ref, mem_ref, wq_ref, wk_ref, wv_ref,
                   r_ref, alpha_ref, idx_ref, scores_ref):
    f32 = jnp.float32
    bf = jnp.bfloat16
    hp = jax.lax.Precision.HIGHEST
    q = jax.lax.dot_general(u_ref[...].astype(bf), wq_ref[...].astype(bf),
                            (((1,), (0,)), ((), ())),
                            preferred_element_type=f32)
    qb = q.astype(bf)
    qrep = jnp.broadcast_to(qb[:, None, :], (BT, H, D_Q)).reshape(R, D_Q)
    hrow = jax.lax.broadcasted_iota(jnp.int32, (R, D_Q), 0) % H
    hcol = jax.lax.broadcasted_iota(jnp.int32, (R, D_Q), 1) // HD
    qexp = jnp.where(hrow == hcol, qrep, jnp.zeros((), bf))

    wk_b = wk_ref[...].astype(bf)
    scale = 1.0 / (HD ** 0.5)
    for t in range(BT):
        k_t = jax.lax.dot_general(mem_ref[t].astype(bf), wk_b,
                                  (((1,), (0,)), ((), ())),
                                  preferred_element_type=f32)
        s_t = jax.lax.dot_general(qexp[t * H:(t + 1) * H, :], k_t.astype(bf),
                                  (((1,), (1,)), ((), ())),
                                  preferred_element_type=f32)
        s_t = s_t * scale - LOAD_PENALTY * ema_ref[t:t + 1, :]
        scores_ref[t * H:(t + 1) * H, :] = s_t

    s = scores_ref[...]
    lane = jax.lax.broadcasted_iota(jnp.int32, (R, J), 1)
    ts_cols = []
    ix_cols = []
    wsel = jnp.zeros((R, J), dtype=f32)
    m0 = None
    for i in range(KAPPA):
        m = jnp.max(s, axis=1, keepdims=True)
        if i == 0:
            m0 = m
        ix = jnp.min(jnp.where(s == m, lane, J), axis=1, keepdims=True)
        ts_cols.append(m)
        ix_cols.append(ix)
        e_i = jnp.exp(m - m0)
        hit = lane == ix
        wsel = wsel + jnp.where(hit, e_i, 0.0)
        s = jnp.where(hit, -jnp.inf, s)
    ts = jnp.concatenate(ts_cols, axis=1)
    ixs = jnp.concatenate(ix_cols, axis=1)

    e = jnp.exp(ts - ts[:, 0:1])
    esum = jnp.sum(e, axis=1, keepdims=True)
    a = e / esum

    alpha_ref[...] = a
    idx_ref[...] = ixs
    wsel = wsel * (1.0 / esum)

    mam_cols = []
    for t in range(BT):
        mam_t = jax.lax.dot_general(wsel[t * H:(t + 1) * H, :], mem_ref[t],
                                    (((1,), (0,)), ((), ())),
                                    preferred_element_type=f32,
                            precision=jax.lax.Precision.HIGHEST)
        mam_cols.append(mam_t)
    mam = jnp.concatenate(mam_cols, axis=0)
    full = jax.lax.dot_general(mam, wv_ref[...], (((1,), (0,)), ((), ())),
                               preferred_element_type=f32,
                            precision=jax.lax.Precision.HIGHEST)
    vrow = jax.lax.broadcasted_iota(jnp.int32, (R, D_V), 0) % H
    vcol = jax.lax.broadcasted_iota(jnp.int32, (R, D_V), 1) // VHD
    full = jnp.where(vrow == vcol, full, 0.0)
    grow = jax.lax.broadcasted_iota(jnp.int32, (BT, R), 0)
    gcol = jax.lax.broadcasted_iota(jnp.int32, (BT, R), 1) // H
    g = jnp.where(grow == gcol, 1.0, 0.0).astype(f32)
    r_ref[...] = jax.lax.dot_general(g, full, (((1,), (0,)), ((), ())),
                                     preferred_element_type=f32,
                            precision=jax.lax.Precision.HIGHEST)


@jax.jit
def kernel(u_t, memory_state, read_ema, W_q, W_k, W_v):
    grid = (B // BT,)
    out_shapes = (
        jax.ShapeDtypeStruct((B, D_V), jnp.float32),
        jax.ShapeDtypeStruct((B * H, KAPPA), jnp.float32),
        jax.ShapeDtypeStruct((B * H, KAPPA), jnp.int32),
    )
    in_specs = [
        pl.BlockSpec((BT, INPUT_DIM), lambda i: (i, 0)),
        pl.BlockSpec((BT, J), lambda i: (i, 0)),
        pl.BlockSpec((BT, J, D_M), lambda i: (i, 0, 0)),
        pl.BlockSpec((INPUT_DIM, D_Q), lambda i: (0, 0)),
        pl.BlockSpec((D_M, D_Q), lambda i: (0, 0)),
        pl.BlockSpec((D_M, D_V), lambda i: (0, 0)),
    ]
    out_specs = (
        pl.BlockSpec((BT, D_V), lambda i: (i, 0)),
        pl.BlockSpec((R, KAPPA), lambda i: (i, 0)),
        pl.BlockSpec((R, KAPPA), lambda i: (i, 0)),
    )
    r_t, alpha2, idx2 = pl.pallas_call(
        _router_kernel,
        grid=grid,
        in_specs=in_specs,
        out_specs=out_specs,
        out_shape=out_shapes,
        scratch_shapes=[pltpu.VMEM((R, J), jnp.float32)],
    )(u_t, read_ema, memory_state, W_q, W_k, W_v)
    return (r_t, alpha2.reshape(B, H, KAPPA), idx2.reshape(B, H, KAPPA))


# speedup vs baseline: 4.4758x; 1.2682x over previous
import functools

import jax
import jax.numpy as jnp
from jax.experimental import pallas as pl
from jax.experimental.pallas import tpu as pltpu

B = 1024
J = 2048
INPUT_DIM = 1792
D_M = 64
H = 8
HD = 16
D_Q = H * HD
D_V = 64
VHD = D_V // H
KAPPA = 8
LOAD_PENALTY = 0.01

BT = 16
R = BT * H


def _router_kernel(u_ref, ema_ref, mem_ref, wq_ref, wk_ref, wv_ref,
                   r_ref, alpha_ref, idx_ref, scores_ref):
    f32 = jnp.float32
    bf = jnp.bfloat16
    hp = jax.lax.Precision.HIGHEST
    q = jax.lax.dot_general(u_ref[...].astype(bf), wq_ref[...].astype(bf),
                            (((1,), (0,)), ((), ())),
                            preferred_element_type=f32)
    qb = q.astype(bf)
    qrep = jnp.broadcast_to(qb[:, None, :], (BT, H, D_Q)).reshape(R, D_Q)
    hrow = jax.lax.broadcasted_iota(jnp.int32, (R, D_Q), 0) % H
    hcol = jax.lax.broadcasted_iota(jnp.int32, (R, D_Q), 1) // HD
    qexp = jnp.where(hrow == hcol, qrep, jnp.zeros((), bf))

    wk_b = wk_ref[...].astype(bf)
    scale = 1.0 / (HD ** 0.5)
    for t in range(BT):
        k_t = jax.lax.dot_general(mem_ref[t].astype(bf), wk_b,
                                  (((1,), (0,)), ((), ())),
                                  preferred_element_type=f32)
        s_t = jax.lax.dot_general(qexp[t * H:(t + 1) * H, :], k_t.astype(bf),
                                  (((1,), (1,)), ((), ())),
                                  preferred_element_type=f32)
        s_t = s_t * scale - LOAD_PENALTY * ema_ref[t:t + 1, :]
        scores_ref[t * H:(t + 1) * H, :] = s_t

    s = scores_ref[...]
    lane = jax.lax.broadcasted_iota(jnp.int32, (R, J), 1)
    ts_cols = []
    ix_cols = []
    wsel = jnp.zeros((R, J), dtype=f32)
    m0 = None
    for i in range(KAPPA):
        m = jnp.max(s, axis=1, keepdims=True)
        if i == 0:
            m0 = m
        ix = jnp.min(jnp.where(s == m, lane, J), axis=1, keepdims=True)
        ts_cols.append(m)
        ix_cols.append(ix)
        e_i = jnp.exp(m - m0)
        hit = lane == ix
        wsel = wsel + jnp.where(hit, e_i, 0.0)
        s = jnp.where(hit, -jnp.inf, s)
    ts = jnp.concatenate(ts_cols, axis=1)
    ixs = jnp.concatenate(ix_cols, axis=1)

    e = jnp.exp(ts - ts[:, 0:1])
    esum = jnp.sum(e, axis=1, keepdims=True)
    a = e / esum

    alpha_ref[...] = a
    idx_ref[...] = ixs
    wsel = wsel * (1.0 / esum)

    wsel_b = wsel.astype(bf)
    mam_cols = []
    for t in range(BT):
        mam_t = jax.lax.dot_general(wsel_b[t * H:(t + 1) * H, :],
                                    mem_ref[t].astype(bf),
                                    (((1,), (0,)), ((), ())),
                                    preferred_element_type=f32)
        mam_cols.append(mam_t)
    mam = jnp.concatenate(mam_cols, axis=0)
    full = jax.lax.dot_general(mam, wv_ref[...], (((1,), (0,)), ((), ())),
                               preferred_element_type=f32,
                            precision=jax.lax.Precision.HIGHEST)
    vrow = jax.lax.broadcasted_iota(jnp.int32, (R, D_V), 0) % H
    vcol = jax.lax.broadcasted_iota(jnp.int32, (R, D_V), 1) // VHD
    full = jnp.where(vrow == vcol, full, 0.0)
    grow = jax.lax.broadcasted_iota(jnp.int32, (BT, R), 0)
    gcol = jax.lax.broadcasted_iota(jnp.int32, (BT, R), 1) // H
    g = jnp.where(grow == gcol, 1.0, 0.0).astype(f32)
    r_ref[...] = jax.lax.dot_general(g, full, (((1,), (0,)), ((), ())),
                                     preferred_element_type=f32,
                            precision=jax.lax.Precision.HIGHEST)


@jax.jit
def kernel(u_t, memory_state, read_ema, W_q, W_k, W_v):
    grid = (B // BT,)
    out_shapes = (
        jax.ShapeDtypeStruct((B, D_V), jnp.float32),
        jax.ShapeDtypeStruct((B * H, KAPPA), jnp.float32),
        jax.ShapeDtypeStruct((B * H, KAPPA), jnp.int32),
    )
    in_specs = [
        pl.BlockSpec((BT, INPUT_DIM), lambda i: (i, 0)),
        pl.BlockSpec((BT, J), lambda i: (i, 0)),
        pl.BlockSpec((BT, J, D_M), lambda i: (i, 0, 0)),
        pl.BlockSpec((INPUT_DIM, D_Q), lambda i: (0, 0)),
        pl.BlockSpec((D_M, D_Q), lambda i: (0, 0)),
        pl.BlockSpec((D_M, D_V), lambda i: (0, 0)),
    ]
    out_specs = (
        pl.BlockSpec((BT, D_V), lambda i: (i, 0)),
        pl.BlockSpec((R, KAPPA), lambda i: (i, 0)),
        pl.BlockSpec((R, KAPPA), lambda i: (i, 0)),
    )
    r_t, alpha2, idx2 = pl.pallas_call(
        _router_kernel,
        grid=grid,
        in_specs=in_specs,
        out_specs=out_specs,
        out_shape=out_shapes,
        scratch_shapes=[pltpu.VMEM((R, J), jnp.float32)],
    )(u_t, read_ema, memory_state, W_q, W_k, W_v)
    return (r_t, alpha2.reshape(B, H, KAPPA), idx2.reshape(B, H, KAPPA))


# speedup vs baseline: 4.8959x; 1.0939x over previous
import jax
import jax.numpy as jnp
from jax.experimental import pallas as pl
from jax.experimental.pallas import tpu as pltpu

B = 1024
J = 2048
INPUT_DIM = 1792
D_M = 64
H = 8
HD = 16
D_Q = H * HD
D_V = 64
VHD = D_V // H
KAPPA = 8
LOAD_PENALTY = 0.01

BT = 16
R = BT * H
NBUF = 3
GRID = B // BT


def _router_kernel(u_ref, ema_ref, mem_hbm, wq_ref, wk_ref, wv_ref,
                   r_ref, alpha_ref, idx_ref, scores_ref, buf, sem):
    f32 = jnp.float32
    i = pl.program_id(0)
    slot = jax.lax.rem(i, NBUF)

    @pl.when(i == 0)
    def _prologue():
        for b in range(NBUF):
            pltpu.make_async_copy(mem_hbm.at[pl.ds(b * BT, BT)],
                                  buf.at[b], sem.at[b]).start()

    pltpu.make_async_copy(mem_hbm.at[pl.ds(i * BT, BT)],
                          buf.at[slot], sem.at[slot]).wait()

    q = jax.lax.dot_general(u_ref[...], wq_ref[...],
                            (((1,), (0,)), ((), ())),
                            preferred_element_type=f32)
    qrep = jnp.broadcast_to(q[:, None, :], (BT, H, D_Q)).reshape(R, D_Q)
    hrow = jax.lax.broadcasted_iota(jnp.int32, (R, D_Q), 0) % H
    hcol = jax.lax.broadcasted_iota(jnp.int32, (R, D_Q), 1) // HD
    bf = jnp.bfloat16
    qb = q.astype(bf)
    qrep_b = jnp.broadcast_to(qb[:, None, :], (BT, H, D_Q)).reshape(R, D_Q)
    qexp = jnp.where(hrow == hcol, qrep_b, jnp.zeros((), bf))

    wk = wk_ref[...]
    scale = 1.0 / (HD ** 0.5)
    TG = 4
    for g in range(BT // TG):
        mem_g = buf[slot, g * TG:(g + 1) * TG].reshape(TG * J, D_M)
        k_g = jax.lax.dot_general(mem_g, wk, (((1,), (0,)), ((), ())),
                                  preferred_element_type=f32)
        k_gb = k_g.astype(bf)
        for tt in range(TG):
            t = g * TG + tt
            s_t = jax.lax.dot_general(qexp[t * H:(t + 1) * H, :],
                                      k_gb[tt * J:(tt + 1) * J, :],
                                      (((1,), (1,)), ((), ())),
                                      preferred_element_type=f32)
            s_t = s_t * scale - LOAD_PENALTY * ema_ref[t:t + 1, :]
            scores_ref[t * H:(t + 1) * H, :] = s_t

    s = scores_ref[...]
    lane = jax.lax.broadcasted_iota(jnp.int32, (R, J), 1)
    ts_cols = []
    ix_cols = []
    for kk in range(KAPPA):
        m = jnp.max(s, axis=1, keepdims=True)
        ix = jnp.min(jnp.where(s == m, lane, J), axis=1, keepdims=True)
        ts_cols.append(m)
        ix_cols.append(ix)
        if kk < KAPPA - 1:
            s = jnp.where(lane == ix, -jnp.inf, s)
    ts = jnp.concatenate(ts_cols, axis=1)
    ixs = jnp.concatenate(ix_cols, axis=1)
    m0 = ts_cols[0]
    wsel = jnp.where(lane == ix_cols[0], 1.0, 0.0)
    for kk in range(1, KAPPA):
        e_i = jnp.exp(ts_cols[kk] - m0)
        wsel = wsel + jnp.where(lane == ix_cols[kk], e_i, 0.0)

    e = jnp.exp(ts - ts[:, 0:1])
    esum = jnp.sum(e, axis=1, keepdims=True)
    a = e / esum

    alpha_ref[...] = a
    idx_ref[...] = ixs
    wsel = wsel * (1.0 / esum)

    mam_cols = []
    for t in range(BT):
        mam_t = jax.lax.dot_general(wsel[t * H:(t + 1) * H, :],
                                    buf[slot, t],
                                    (((1,), (0,)), ((), ())),
                                    preferred_element_type=f32)
        mam_cols.append(mam_t)
    mam = jnp.concatenate(mam_cols, axis=0)
    full = jax.lax.dot_general(mam, wv_ref[...], (((1,), (0,)), ((), ())),
                               preferred_element_type=f32)
    vrow = jax.lax.broadcasted_iota(jnp.int32, (R, D_V), 0) % H
    vcol = jax.lax.broadcasted_iota(jnp.int32, (R, D_V), 1) // VHD
    full = jnp.where(vrow == vcol, full, 0.0)
    grow = jax.lax.broadcasted_iota(jnp.int32, (BT, R), 0)
    gcol = jax.lax.broadcasted_iota(jnp.int32, (BT, R), 1) // H
    g = jnp.where(grow == gcol, 1.0, 0.0).astype(f32)
    r_ref[...] = jax.lax.dot_general(g, full, (((1,), (0,)), ((), ())),
                                     preferred_element_type=f32)

    @pl.when(i + NBUF < GRID)
    def _refill():
        pltpu.make_async_copy(mem_hbm.at[pl.ds((i + NBUF) * BT, BT)],
                              buf.at[slot], sem.at[slot]).start()


@jax.jit
def kernel(u_t, memory_state, read_ema, W_q, W_k, W_v):
    out_shapes = (
        jax.ShapeDtypeStruct((B, D_V), jnp.float32),
        jax.ShapeDtypeStruct((B * H, KAPPA), jnp.float32),
        jax.ShapeDtypeStruct((B * H, KAPPA), jnp.int32),
    )
    in_specs = [
        pl.BlockSpec((BT, INPUT_DIM), lambda i: (i, 0)),
        pl.BlockSpec((BT, J), lambda i: (i, 0)),
        pl.BlockSpec(memory_space=pl.ANY),
        pl.BlockSpec((INPUT_DIM, D_Q), lambda i: (0, 0)),
        pl.BlockSpec((D_M, D_Q), lambda i: (0, 0)),
        pl.BlockSpec((D_M, D_V), lambda i: (0, 0)),
    ]
    out_specs = (
        pl.BlockSpec((BT, D_V), lambda i: (i, 0)),
        pl.BlockSpec((R, KAPPA), lambda i: (i, 0)),
        pl.BlockSpec((R, KAPPA), lambda i: (i, 0)),
    )
    r_t, alpha2, idx2 = pl.pallas_call(
        _router_kernel,
        grid=(GRID,),
        in_specs=in_specs,
        out_specs=out_specs,
        out_shape=out_shapes,
        scratch_shapes=[
            pltpu.VMEM((R, J), jnp.float32),
            pltpu.VMEM((NBUF, BT, J, D_M), jnp.float32),
            pltpu.SemaphoreType.DMA((NBUF,)),
        ],
    )(u_t, read_ema, memory_state, W_q, W_k, W_v)
    return (r_t, alpha2.reshape(B, H, KAPPA), idx2.reshape(B, H, KAPPA))
